# baseline (speedup 1.0000x reference)
import functools

import jax
import jax.numpy as jnp
from jax import lax
from jax.experimental import pallas as pl
from jax.experimental.pallas import tpu as pltpu
from jax.experimental.pallas import tpu_sc as plsc

N = 10000
D_IN = 128
D_H = 64
K = 6

NC = 2
NS = 16
NW = NC * NS
CHUNK = 128
DEPTH = 3
LANES = 16

ACC_ROWS = 10112
RPT = ACC_ROWS // NS
TRASH = N
TRASH_SPREAD = 96

_MESH = plsc.VectorSubcoreMesh(
    core_axis_name="c", subcore_axis_name="s", num_cores=NC, num_subcores=NS
)


def _tile_id():
    ci = lax.axis_index("c")
    sid = lax.axis_index("s")
    return ci, sid, sid * NC + ci


def _edge_prep(src2d, dst2d):
    rows = src2d.shape[0]
    blk = 256

    def body(s_ref, d_ref, se_ref, de_ref):
        s = s_ref[...]
        d = d_ref[...]
        m = s == d
        trash = TRASH + lax.broadcasted_iota(jnp.int32, (blk, CHUNK), 1) % TRASH_SPREAD
        se_ref[...] = jnp.where(m, trash, s)
        de_ref[...] = jnp.where(m, trash, d)

    return pl.pallas_call(
        body,
        grid=(rows // blk,),
        in_specs=[pl.BlockSpec((blk, CHUNK), lambda i: (i, 0)),
                  pl.BlockSpec((blk, CHUNK), lambda i: (i, 0))],
        out_specs=[pl.BlockSpec((blk, CHUNK), lambda i: (i, 0)),
                   pl.BlockSpec((blk, CHUNK), lambda i: (i, 0))],
        out_shape=[jax.ShapeDtypeStruct((rows, CHUNK), jnp.int32),
                   jax.ShapeDtypeStruct((rows, CHUNK), jnp.int32)],
    )(src2d, dst2d)


def _make_matvec_kernel(nch):
    @functools.partial(
        pl.kernel,
        out_type=jax.ShapeDtypeStruct((NC, ACC_ROWS, D_IN), jnp.float32),
        mesh=_MESH,
        scratch_types=[
            pltpu.VMEM_SHARED((ACC_ROWS, D_IN), jnp.float32),
            [pltpu.VMEM((1, CHUNK), jnp.int32) for _ in range(DEPTH)],
            [pltpu.VMEM((1, CHUNK), jnp.int32) for _ in range(DEPTH)],
            [pltpu.VMEM((CHUNK, D_IN), jnp.float32) for _ in range(DEPTH)],
            [pltpu.SemaphoreType.DMA for _ in range(3 * DEPTH)],
        ],
    )
    def matvec_kernel(p_hbm, src_hbm, dst_hbm, zerosd_hbm, u_hbm,
                      acc, sbis, dbis, rows, sems):
        ci, sid, w = _tile_id()
        pltpu.sync_copy(zerosd_hbm.at[pl.ds(sid * RPT, RPT)],
                        acc.at[pl.ds(sid * RPT, RPT)])
        plsc.subcore_barrier()

        def do_group(c0, nb):
            idxd = []
            for i in range(nb):
                g = w * nch + c0 + i
                dS = pltpu.async_copy(src_hbm.at[g], sbis[i], sems[3 * i])
                dD = pltpu.async_copy(dst_hbm.at[g], dbis[i], sems[3 * i + 1])
                idxd.append((dS, dD))
            gd = []
            for i in range(nb):
                idxd[i][0].wait()
                gd.append(pltpu.async_copy(p_hbm.at[sbis[i].at[0]], rows[i],
                                           sems[3 * i + 2]))
            for i in range(nb):
                gd[i].wait()
                idxd[i][1].wait()
                pltpu.sync_copy(rows[i], acc.at[dbis[i].at[0]], add=True)

        ngrp = nch // DEPTH
        rem = nch - ngrp * DEPTH

        @pl.loop(0, ngrp)
        def _(q):
            do_group(q * DEPTH, DEPTH)

        if rem:
            do_group(ngrp * DEPTH, rem)

        plsc.subcore_barrier()
        pltpu.sync_copy(acc.at[pl.ds(sid * RPT, RPT)],
                        u_hbm.at[ci, pl.ds(sid * RPT, RPT)])

    return matvec_kernel


_RB = 1000
_GRID = (N // _RB,)


def _step0(x, degtab, w0):
    def body(x_ref, d0_ref, d1_ref, w_ref, g_ref, p_ref, o_ref):
        deg = d0_ref[0, :, 0:1] + d1_ref[0, :, 0:1]
        g = jnp.where(deg > 0.0, lax.rsqrt(jnp.maximum(deg, 1e-30)), 0.0)
        xb = x_ref[...]
        g_ref[...] = jnp.broadcast_to(g, (_RB, D_IN))
        p_ref[...] = g * xb
        o_ref[...] = jnp.dot(xb, w_ref[...], preferred_element_type=jnp.float32)

    return pl.pallas_call(
        body,
        grid=_GRID,
        in_specs=[
            pl.BlockSpec((_RB, D_IN), lambda i: (i, 0)),
            pl.BlockSpec((1, _RB, D_IN), lambda i: (0, i, 0)),
            pl.BlockSpec((1, _RB, D_IN), lambda i: (1, i, 0)),
            pl.BlockSpec((D_IN, D_H), lambda i: (0, 0)),
        ],
        out_specs=[
            pl.BlockSpec((_RB, D_IN), lambda i: (i, 0)),
            pl.BlockSpec((_RB, D_IN), lambda i: (i, 0)),
            pl.BlockSpec((_RB, D_H), lambda i: (i, 0)),
        ],
        out_shape=[
            jax.ShapeDtypeStruct((N, D_IN), jnp.float32),
            jax.ShapeDtypeStruct((N, D_IN), jnp.float32),
            jax.ShapeDtypeStruct((N, D_H), jnp.float32),
        ],
    )(x, degtab, degtab, w0)


def _step_k(u, g, tprev, wk, out_in, c, has_prev, is_last, bias=None):
    def body(*refs):
        if is_last:
            u0_ref, u1_ref, g_ref, tp_ref, w_ref, b_ref, oi_ref, o_ref = refs
        else:
            u0_ref, u1_ref, g_ref, tp_ref, w_ref, oi_ref, t_ref, p_ref, o_ref = refs
        ub = u0_ref[0] + u1_ref[0]
        t = (-float(c)) * g_ref[...] * ub
        if has_prev:
            t = t - tp_ref[...]
        acc = oi_ref[...] + jnp.dot(t, refs[4][...],
                                    preferred_element_type=jnp.float32)
        if is_last:
            o_ref[...] = jnp.maximum(acc + b_ref[...], 0.0)
        else:
            t_ref[...] = t
            p_ref[...] = g_ref[...] * t
            o_ref[...] = acc

    in_specs = [
        pl.BlockSpec((1, _RB, D_IN), lambda i: (0, i, 0)),
        pl.BlockSpec((1, _RB, D_IN), lambda i: (1, i, 0)),
        pl.BlockSpec((_RB, D_IN), lambda i: (i, 0)),
        pl.BlockSpec((_RB, D_IN), lambda i: (i, 0)),
        pl.BlockSpec((D_IN, D_H), lambda i: (0, 0)),
    ]
    args = [u, u, g, tprev, wk]
    if is_last:
        in_specs.append(pl.BlockSpec((1, D_H), lambda i: (0, 0)))
        args.append(bias)
    in_specs.append(pl.BlockSpec((_RB, D_H), lambda i: (i, 0)))
    args.append(out_in)

    out_specs = []
    out_shape = []
    if not is_last:
        out_specs += [pl.BlockSpec((_RB, D_IN), lambda i: (i, 0)),
                      pl.BlockSpec((_RB, D_IN), lambda i: (i, 0))]
        out_shape += [jax.ShapeDtypeStruct((N, D_IN), jnp.float32),
                      jax.ShapeDtypeStruct((N, D_IN), jnp.float32)]
    out_specs.append(pl.BlockSpec((_RB, D_H), lambda i: (i, 0)))
    out_shape.append(jax.ShapeDtypeStruct((N, D_H), jnp.float32))

    n_in = len(args)
    alias = {n_in - 1: len(out_shape) - 1}

    return pl.pallas_call(
        body,
        grid=_GRID,
        in_specs=in_specs,
        out_specs=out_specs,
        out_shape=out_shape,
        input_output_aliases=alias,
    )(*args)


def kernel(x, edge_index, W, b):
    E = edge_index.shape[1]
    per_tile = NW * CHUNK
    nch = -(-E // per_tile)
    nch = -(-nch // 8) * 8
    e_pad = nch * per_tile

    src = edge_index[0].astype(jnp.int32)
    dst = edge_index[1].astype(jnp.int32)
    pad = e_pad - E
    if pad:
        z = (jnp.arange(pad, dtype=jnp.int32) * 41) % N
        src = jnp.concatenate([src, z])
        dst = jnp.concatenate([dst, z])
    src2d = src.reshape(NW * nch, CHUNK)
    dst2d = dst.reshape(NW * nch, CHUNK)

    zerosd = jnp.zeros((ACC_ROWS, D_IN), jnp.float32)
    ones_tab = jnp.ones((2048, D_IN), jnp.float32)
    zidx3d = ((jnp.arange(e_pad, dtype=jnp.int32) * 997) % 2048).reshape(
        NW * nch, 1, CHUNK)

    seff2d, deff2d = _edge_prep(src2d, dst2d)
    seff3d = seff2d.reshape(NW * nch, 1, CHUNK)
    deff3d = deff2d.reshape(NW * nch, 1, CHUNK)
    src3d = src2d.reshape(NW * nch, 1, CHUNK)

    matvec = _make_matvec_kernel(nch)
    degu = matvec(ones_tab, zidx3d, seff3d, zerosd)

    g, p, out = _step0(x, degu, W[0])
    tprev = x
    tcur = None
    for k in range(1, K):
        u = matvec(p, src3d, deff3d, zerosd)
        is_last = k == K - 1
        c = 1 if k == 1 else 2
        if is_last:
            (out,) = _step_k(u, g, tprev, W[k], out, c=c, has_prev=True,
                             is_last=True, bias=b.reshape(1, D_H))
        elif k == 1:
            tcur, p, out = _step_k(u, g, x, W[k], out, c=1, has_prev=False,
                                   is_last=False)
            tprev = x
        else:
            tnew, p, out = _step_k(u, g, tprev, W[k], out, c=2, has_prev=True,
                                   is_last=False)
            tprev = tcur
            tcur = tnew
    return out


# speedup vs baseline: 1.0306x; 1.0306x over previous
import functools

import jax
import jax.numpy as jnp
from jax import lax
from jax.experimental import pallas as pl
from jax.experimental.pallas import tpu as pltpu
from jax.experimental.pallas import tpu_sc as plsc

N = 10000
D_IN = 128
D_H = 64
K = 6

NC = 2
NS = 16
NW = NC * NS
CHUNK = 128
LANES = 16

ACC_ROWS = 10240
RPT = ACC_ROWS // NS
TRASH = N

_MESH = plsc.VectorSubcoreMesh(
    core_axis_name="c", subcore_axis_name="s", num_cores=NC, num_subcores=NS
)


def _tile_id():
    ci = lax.axis_index("c")
    sid = lax.axis_index("s")
    return ci, sid, sid * NC + ci


def _edge_prep(src2d, dst2d):
    rows = src2d.shape[0]
    blk = 256

    def body(s_ref, d_ref, se_ref, de_ref):
        s = s_ref[...]
        d = d_ref[...]
        m = s == d
        trash = TRASH + lax.broadcasted_iota(jnp.int32, (blk, CHUNK), 1)
        se_ref[...] = jnp.where(m, trash, s)
        de_ref[...] = jnp.where(m, trash, d)

    return pl.pallas_call(
        body,
        grid=(rows // blk,),
        in_specs=[pl.BlockSpec((blk, CHUNK), lambda i: (i, 0)),
                  pl.BlockSpec((blk, CHUNK), lambda i: (i, 0))],
        out_specs=[pl.BlockSpec((blk, CHUNK), lambda i: (i, 0)),
                   pl.BlockSpec((blk, CHUNK), lambda i: (i, 0))],
        out_shape=[jax.ShapeDtypeStruct((rows, CHUNK), jnp.int32),
                   jax.ShapeDtypeStruct((rows, CHUNK), jnp.int32)],
    )(src2d, dst2d)


def _make_matvec_kernel(nch):
    @functools.partial(
        pl.kernel,
        out_type=jax.ShapeDtypeStruct((NC, ACC_ROWS, D_IN), jnp.float32),
        mesh=_MESH,
        scratch_types=[
            pltpu.VMEM_SHARED((ACC_ROWS, D_IN), jnp.float32),
            pltpu.VMEM((nch, 1, CHUNK), jnp.int32),
            pltpu.VMEM((1, CHUNK), jnp.int32),
            pltpu.VMEM((1, CHUNK), jnp.int32),
            pltpu.VMEM((CHUNK, D_IN), jnp.float32),
            pltpu.VMEM((CHUNK, D_IN), jnp.float32),
            pltpu.SemaphoreType.DMA,
            pltpu.SemaphoreType.DMA,
            pltpu.SemaphoreType.DMA,
            pltpu.SemaphoreType.DMA,
        ],
    )
    def matvec_kernel(p_hbm, src_hbm, dst_hbm, zerosd_hbm, u_hbm,
                      acc, sbuf, dbi0, dbi1, rows0, rows1,
                      sem0, sem1, semd0, semd1):
        ci, sid, w = _tile_id()
        pltpu.sync_copy(zerosd_hbm.at[pl.ds(sid * RPT, RPT)],
                        acc.at[pl.ds(sid * RPT, RPT)])
        pltpu.sync_copy(src_hbm.at[pl.ds(w * nch, nch)], sbuf)
        plsc.subcore_barrier()

        @pl.loop(0, nch // 2)
        def _(pr):
            cA = pr * 2
            cB = cA + 1
            gA = w * nch + cA
            gB = gA + 1
            dDA = pltpu.async_copy(dst_hbm.at[gA], dbi0, semd0)
            dDB = pltpu.async_copy(dst_hbm.at[gB], dbi1, semd1)
            dA = pltpu.async_copy(p_hbm.at[sbuf.at[cA, 0]], rows0, sem0)
            dB = pltpu.async_copy(p_hbm.at[sbuf.at[cB, 0]], rows1, sem1)
            dA.wait()
            dDA.wait()
            pltpu.sync_copy(rows0, acc.at[dbi0.at[0]], add=True)
            dB.wait()
            dDB.wait()
            pltpu.sync_copy(rows1, acc.at[dbi1.at[0]], add=True)

        plsc.subcore_barrier()
        pltpu.sync_copy(acc.at[pl.ds(sid * RPT, RPT)],
                        u_hbm.at[ci, pl.ds(sid * RPT, RPT)])

    return matvec_kernel


_RB = 1000
_GRID = (N // _RB,)


def _step0(x, degtab, w0):
    def body(x_ref, d0_ref, d1_ref, w_ref, g_ref, p_ref, o_ref):
        deg = d0_ref[0, :, 0:1] + d1_ref[0, :, 0:1]
        g = jnp.where(deg > 0.0, lax.rsqrt(jnp.maximum(deg, 1e-30)), 0.0)
        xb = x_ref[...]
        g_ref[...] = jnp.broadcast_to(g, (_RB, D_IN))
        p_ref[...] = g * xb
        o_ref[...] = jnp.dot(xb, w_ref[...], preferred_element_type=jnp.float32)

    return pl.pallas_call(
        body,
        grid=_GRID,
        in_specs=[
            pl.BlockSpec((_RB, D_IN), lambda i: (i, 0)),
            pl.BlockSpec((1, _RB, D_IN), lambda i: (0, i, 0)),
            pl.BlockSpec((1, _RB, D_IN), lambda i: (1, i, 0)),
            pl.BlockSpec((D_IN, D_H), lambda i: (0, 0)),
        ],
        out_specs=[
            pl.BlockSpec((_RB, D_IN), lambda i: (i, 0)),
            pl.BlockSpec((_RB, D_IN), lambda i: (i, 0)),
            pl.BlockSpec((_RB, D_H), lambda i: (i, 0)),
        ],
        out_shape=[
            jax.ShapeDtypeStruct((N, D_IN), jnp.float32),
            jax.ShapeDtypeStruct((N, D_IN), jnp.float32),
            jax.ShapeDtypeStruct((N, D_H), jnp.float32),
        ],
    )(x, degtab, degtab, w0)


def _step_k(u, g, tprev, wk, out_in, c, has_prev, is_last, bias=None):
    def body(*refs):
        if is_last:
            u0_ref, u1_ref, g_ref, tp_ref, w_ref, b_ref, oi_ref, o_ref = refs
        else:
            u0_ref, u1_ref, g_ref, tp_ref, w_ref, oi_ref, t_ref, p_ref, o_ref = refs
        ub = u0_ref[0] + u1_ref[0]
        t = (-float(c)) * g_ref[...] * ub
        if has_prev:
            t = t - tp_ref[...]
        acc = oi_ref[...] + jnp.dot(t, refs[4][...],
                                    preferred_element_type=jnp.float32)
        if is_last:
            o_ref[...] = jnp.maximum(acc + b_ref[...], 0.0)
        else:
            t_ref[...] = t
            p_ref[...] = g_ref[...] * t
            o_ref[...] = acc

    in_specs = [
        pl.BlockSpec((1, _RB, D_IN), lambda i: (0, i, 0)),
        pl.BlockSpec((1, _RB, D_IN), lambda i: (1, i, 0)),
        pl.BlockSpec((_RB, D_IN), lambda i: (i, 0)),
        pl.BlockSpec((_RB, D_IN), lambda i: (i, 0)),
        pl.BlockSpec((D_IN, D_H), lambda i: (0, 0)),
    ]
    args = [u, u, g, tprev, wk]
    if is_last:
        in_specs.append(pl.BlockSpec((1, D_H), lambda i: (0, 0)))
        args.append(bias)
    in_specs.append(pl.BlockSpec((_RB, D_H), lambda i: (i, 0)))
    args.append(out_in)

    out_specs = []
    out_shape = []
    if not is_last:
        out_specs += [pl.BlockSpec((_RB, D_IN), lambda i: (i, 0)),
                      pl.BlockSpec((_RB, D_IN), lambda i: (i, 0))]
        out_shape += [jax.ShapeDtypeStruct((N, D_IN), jnp.float32),
                      jax.ShapeDtypeStruct((N, D_IN), jnp.float32)]
    out_specs.append(pl.BlockSpec((_RB, D_H), lambda i: (i, 0)))
    out_shape.append(jax.ShapeDtypeStruct((N, D_H), jnp.float32))

    n_in = len(args)
    alias = {n_in - 1: len(out_shape) - 1}

    return pl.pallas_call(
        body,
        grid=_GRID,
        in_specs=in_specs,
        out_specs=out_specs,
        out_shape=out_shape,
        input_output_aliases=alias,
    )(*args)


def kernel(x, edge_index, W, b):
    E = edge_index.shape[1]
    per_tile = NW * CHUNK
    nch = -(-E // per_tile)
    nch = -(-nch // 8) * 8
    e_pad = nch * per_tile

    src = edge_index[0].astype(jnp.int32)
    dst = edge_index[1].astype(jnp.int32)
    pad = e_pad - E
    if pad:
        z = (jnp.arange(pad, dtype=jnp.int32) * 41) % N
        src = jnp.concatenate([src, z])
        dst = jnp.concatenate([dst, z])
    src2d = src.reshape(NW * nch, CHUNK)
    dst2d = dst.reshape(NW * nch, CHUNK)

    zerosd = jnp.zeros((ACC_ROWS, D_IN), jnp.float32)
    ones_tab = jnp.ones((2048, D_IN), jnp.float32)
    zidx3d = ((jnp.arange(e_pad, dtype=jnp.int32) * 997) % 2048).reshape(
        NW * nch, 1, CHUNK)

    seff2d, deff2d = _edge_prep(src2d, dst2d)
    seff3d = seff2d.reshape(NW * nch, 1, CHUNK)
    deff3d = deff2d.reshape(NW * nch, 1, CHUNK)
    src3d = src2d.reshape(NW * nch, 1, CHUNK)

    matvec = _make_matvec_kernel(nch)
    degu = matvec(ones_tab, zidx3d, seff3d, zerosd)

    g, p, out = _step0(x, degu, W[0])
    tprev = x
    tcur = None
    for k in range(1, K):
        u = matvec(p, src3d, deff3d, zerosd)
        is_last = k == K - 1
        c = 1 if k == 1 else 2
        if is_last:
            (out,) = _step_k(u, g, tprev, W[k], out, c=c, has_prev=True,
                             is_last=True, bias=b.reshape(1, D_H))
        elif k == 1:
            tcur, p, out = _step_k(u, g, x, W[k], out, c=1, has_prev=False,
                                   is_last=False)
            tprev = x
        else:
            tnew, p, out = _step_k(u, g, tprev, W[k], out, c=2, has_prev=True,
                                   is_last=False)
            tprev = tcur
            tcur = tnew
    return out


# speedup vs baseline: 1.3427x; 1.3028x over previous
import functools

import jax
import jax.numpy as jnp
from jax import lax
from jax.experimental import pallas as pl
from jax.experimental.pallas import tpu as pltpu
from jax.experimental.pallas import tpu_sc as plsc

N = 10000
D_IN = 128
D_H = 64
K = 6

NC = 2
NS = 16
NW = NC * NS
CHUNK = 128
LANES = 16

ACC_ROWS = 10240
RPT = ACC_ROWS // NS
TRASH = N

_MESH = plsc.VectorSubcoreMesh(
    core_axis_name="c", subcore_axis_name="s", num_cores=NC, num_subcores=NS
)


def _tile_id():
    ci = lax.axis_index("c")
    sid = lax.axis_index("s")
    return ci, sid, sid * NC + ci


def _edge_prep(src2d, dst2d):
    rows = src2d.shape[0]
    blk = 256

    def body(s_ref, d_ref, se_ref, de_ref):
        s = s_ref[...]
        d = d_ref[...]
        m = s == d
        trash = TRASH + lax.broadcasted_iota(jnp.int32, (blk, CHUNK), 1)
        se_ref[...] = jnp.where(m, trash, s)
        de_ref[...] = jnp.where(m, trash, d)

    return pl.pallas_call(
        body,
        grid=(rows // blk,),
        in_specs=[pl.BlockSpec((blk, CHUNK), lambda i: (i, 0)),
                  pl.BlockSpec((blk, CHUNK), lambda i: (i, 0))],
        out_specs=[pl.BlockSpec((blk, CHUNK), lambda i: (i, 0)),
                   pl.BlockSpec((blk, CHUNK), lambda i: (i, 0))],
        out_shape=[jax.ShapeDtypeStruct((rows, CHUNK), jnp.int32),
                   jax.ShapeDtypeStruct((rows, CHUNK), jnp.int32)],
    )(src2d, dst2d)


def _make_matvec_kernel(nch, d):
    @functools.partial(
        pl.kernel,
        out_type=jax.ShapeDtypeStruct((NC, ACC_ROWS, d), jnp.float32),
        mesh=_MESH,
        scratch_types=[
            pltpu.VMEM_SHARED((ACC_ROWS, d), jnp.float32),
            pltpu.VMEM((nch, 1, CHUNK), jnp.int32),
            pltpu.VMEM((1, CHUNK), jnp.int32),
            pltpu.VMEM((1, CHUNK), jnp.int32),
            pltpu.VMEM((CHUNK, d), jnp.float32),
            pltpu.VMEM((CHUNK, d), jnp.float32),
            pltpu.SemaphoreType.DMA,
            pltpu.SemaphoreType.DMA,
            pltpu.SemaphoreType.DMA,
            pltpu.SemaphoreType.DMA,
        ],
    )
    def matvec_kernel(p_hbm, src_hbm, dst_hbm, zerosd_hbm, u_hbm,
                      acc, sbuf, dbi0, dbi1, rows0, rows1,
                      sem0, sem1, semd0, semd1):
        ci, sid, w = _tile_id()
        pltpu.sync_copy(zerosd_hbm.at[pl.ds(sid * RPT, RPT)],
                        acc.at[pl.ds(sid * RPT, RPT)])
        pltpu.sync_copy(src_hbm.at[pl.ds(w * nch, nch)], sbuf)
        plsc.subcore_barrier()

        def fire(c, dbi, rows, semg, semd):
            pltpu.async_copy(dst_hbm.at[w * nch + c], dbi, semd)
            pltpu.async_copy(p_hbm.at[sbuf.at[c, 0]], rows, semg)

        def wait_and_scatter(dbi, rows, semg, semd):
            pltpu.make_async_copy(p_hbm.at[sbuf.at[0, 0]], rows, semg).wait()
            pltpu.make_async_copy(dst_hbm.at[w * nch], dbi, semd).wait()
            pltpu.sync_copy(rows, acc.at[dbi.at[0]], add=True)

        npair = nch // 2
        fire(0, dbi0, rows0, sem0, semd0)

        @pl.loop(0, npair - 1)
        def _(pr):
            cA = pr * 2
            fire(cA + 1, dbi1, rows1, sem1, semd1)
            wait_and_scatter(dbi0, rows0, sem0, semd0)
            fire(cA + 2, dbi0, rows0, sem0, semd0)
            wait_and_scatter(dbi1, rows1, sem1, semd1)

        fire(nch - 1, dbi1, rows1, sem1, semd1)
        wait_and_scatter(dbi0, rows0, sem0, semd0)
        wait_and_scatter(dbi1, rows1, sem1, semd1)

        plsc.subcore_barrier()
        pltpu.sync_copy(acc.at[pl.ds(sid * RPT, RPT)],
                        u_hbm.at[ci, pl.ds(sid * RPT, RPT)])

    return matvec_kernel


_RB = 1000
_GRID = (N // _RB,)


def _step0(x, degtab, w0):
    def body(x_ref, d0_ref, d1_ref, w_ref, g_ref, p_ref, o_ref):
        deg = d0_ref[0, :, 0:1] + d1_ref[0, :, 0:1]
        g = jnp.where(deg > 0.0, lax.rsqrt(jnp.maximum(deg, 1e-30)), 0.0)
        xb = x_ref[...]
        g_ref[...] = jnp.broadcast_to(g, (_RB, D_IN))
        p_ref[...] = g * xb
        o_ref[...] = jnp.dot(xb, w_ref[...], preferred_element_type=jnp.float32)

    return pl.pallas_call(
        body,
        grid=_GRID,
        in_specs=[
            pl.BlockSpec((_RB, D_IN), lambda i: (i, 0)),
            pl.BlockSpec((1, _RB, D_IN), lambda i: (0, i, 0)),
            pl.BlockSpec((1, _RB, D_IN), lambda i: (1, i, 0)),
            pl.BlockSpec((D_IN, D_H), lambda i: (0, 0)),
        ],
        out_specs=[
            pl.BlockSpec((_RB, D_IN), lambda i: (i, 0)),
            pl.BlockSpec((_RB, D_IN), lambda i: (i, 0)),
            pl.BlockSpec((_RB, D_H), lambda i: (i, 0)),
        ],
        out_shape=[
            jax.ShapeDtypeStruct((N, D_IN), jnp.float32),
            jax.ShapeDtypeStruct((N, D_IN), jnp.float32),
            jax.ShapeDtypeStruct((N, D_H), jnp.float32),
        ],
    )(x, degtab, degtab, w0)


def _step_k(u, g, tprev, wk, out_in, c, has_prev, is_last, bias=None):
    def body(*refs):
        if is_last:
            u0_ref, u1_ref, g_ref, tp_ref, w_ref, b_ref, oi_ref, o_ref = refs
        else:
            u0_ref, u1_ref, g_ref, tp_ref, w_ref, oi_ref, t_ref, p_ref, o_ref = refs
        ub = u0_ref[0] + u1_ref[0]
        t = (-float(c)) * g_ref[...] * ub
        if has_prev:
            t = t - tp_ref[...]
        acc = oi_ref[...] + jnp.dot(t, refs[4][...],
                                    preferred_element_type=jnp.float32)
        if is_last:
            o_ref[...] = jnp.maximum(acc + b_ref[...], 0.0)
        else:
            t_ref[...] = t
            p_ref[...] = g_ref[...] * t
            o_ref[...] = acc

    in_specs = [
        pl.BlockSpec((1, _RB, D_IN), lambda i: (0, i, 0)),
        pl.BlockSpec((1, _RB, D_IN), lambda i: (1, i, 0)),
        pl.BlockSpec((_RB, D_IN), lambda i: (i, 0)),
        pl.BlockSpec((_RB, D_IN), lambda i: (i, 0)),
        pl.BlockSpec((D_IN, D_H), lambda i: (0, 0)),
    ]
    args = [u, u, g, tprev, wk]
    if is_last:
        in_specs.append(pl.BlockSpec((1, D_H), lambda i: (0, 0)))
        args.append(bias)
    in_specs.append(pl.BlockSpec((_RB, D_H), lambda i: (i, 0)))
    args.append(out_in)

    out_specs = []
    out_shape = []
    if not is_last:
        out_specs += [pl.BlockSpec((_RB, D_IN), lambda i: (i, 0)),
                      pl.BlockSpec((_RB, D_IN), lambda i: (i, 0))]
        out_shape += [jax.ShapeDtypeStruct((N, D_IN), jnp.float32),
                      jax.ShapeDtypeStruct((N, D_IN), jnp.float32)]
    out_specs.append(pl.BlockSpec((_RB, D_H), lambda i: (i, 0)))
    out_shape.append(jax.ShapeDtypeStruct((N, D_H), jnp.float32))

    n_in = len(args)
    alias = {n_in - 1: len(out_shape) - 1}

    return pl.pallas_call(
        body,
        grid=_GRID,
        in_specs=in_specs,
        out_specs=out_specs,
        out_shape=out_shape,
        input_output_aliases=alias,
    )(*args)


def kernel(x, edge_index, W, b):
    E = edge_index.shape[1]
    per_tile = NW * CHUNK
    nch = -(-E // per_tile)
    nch = -(-nch // 8) * 8
    e_pad = nch * per_tile

    src = edge_index[0].astype(jnp.int32)
    dst = edge_index[1].astype(jnp.int32)
    pad = e_pad - E
    if pad:
        z = (jnp.arange(pad, dtype=jnp.int32) * 41) % N
        src = jnp.concatenate([src, z])
        dst = jnp.concatenate([dst, z])
    src2d = src.reshape(NW * nch, CHUNK)
    dst2d = dst.reshape(NW * nch, CHUNK)

    zerosd = jnp.zeros((ACC_ROWS, D_IN), jnp.float32)
    ones_tab = jnp.ones((2048, D_IN), jnp.float32)
    zidx3d = ((jnp.arange(e_pad, dtype=jnp.int32) * 997) % 2048).reshape(
        NW * nch, 1, CHUNK)

    seff2d, deff2d = _edge_prep(src2d, dst2d)
    seff3d = seff2d.reshape(NW * nch, 1, CHUNK)
    deff3d = deff2d.reshape(NW * nch, 1, CHUNK)
    src3d = src2d.reshape(NW * nch, 1, CHUNK)

    matvec = _make_matvec_kernel(nch, D_IN)
    degu = matvec(ones_tab, zidx3d, seff3d, zerosd)

    g, p, out = _step0(x, degu, W[0])
    tprev = x
    tcur = None
    for k in range(1, K):
        u = matvec(p, src3d, deff3d, zerosd)
        is_last = k == K - 1
        c = 1 if k == 1 else 2
        if is_last:
            (out,) = _step_k(u, g, tprev, W[k], out, c=c, has_prev=True,
                             is_last=True, bias=b.reshape(1, D_H))
        elif k == 1:
            tcur, p, out = _step_k(u, g, x, W[k], out, c=1, has_prev=False,
                                   is_last=False)
            tprev = x
        else:
            tnew, p, out = _step_k(u, g, tprev, W[k], out, c=2, has_prev=True,
                                   is_last=False)
            tprev = tcur
            tcur = tnew
    return out


# speedup vs baseline: 1.4622x; 1.0890x over previous
import functools

import jax
import jax.numpy as jnp
from jax import lax
from jax.experimental import pallas as pl
from jax.experimental.pallas import tpu as pltpu
from jax.experimental.pallas import tpu_sc as plsc

N = 10000
D_IN = 128
D_H = 64
K = 6

NC = 2
NS = 16
NW = NC * NS
CHUNK = 128
LANES = 16

ACC_ROWS = 10240
RPT = ACC_ROWS // NS
TRASH = N

_MESH = plsc.VectorSubcoreMesh(
    core_axis_name="c", subcore_axis_name="s", num_cores=NC, num_subcores=NS
)


def _tile_id():
    ci = lax.axis_index("c")
    sid = lax.axis_index("s")
    return ci, sid, sid * NC + ci


def _edge_prep(src2d, dst2d):
    rows = src2d.shape[0]
    blk = 256

    def body(s_ref, d_ref, se_ref, de_ref):
        s = s_ref[...]
        d = d_ref[...]
        m = s == d
        trash = TRASH + lax.broadcasted_iota(jnp.int32, (blk, CHUNK), 1)
        se_ref[...] = jnp.where(m, trash, s)
        de_ref[...] = jnp.where(m, trash, d)

    return pl.pallas_call(
        body,
        grid=(rows // blk,),
        in_specs=[pl.BlockSpec((blk, CHUNK), lambda i: (i, 0)),
                  pl.BlockSpec((blk, CHUNK), lambda i: (i, 0))],
        out_specs=[pl.BlockSpec((blk, CHUNK), lambda i: (i, 0)),
                   pl.BlockSpec((blk, CHUNK), lambda i: (i, 0))],
        out_shape=[jax.ShapeDtypeStruct((rows, CHUNK), jnp.int32),
                   jax.ShapeDtypeStruct((rows, CHUNK), jnp.int32)],
    )(src2d, dst2d)


def _make_matvec_kernel(nch, d):
    @functools.partial(
        pl.kernel,
        out_type=jax.ShapeDtypeStruct((NC, ACC_ROWS, d), jnp.float32),
        mesh=_MESH,
        scratch_types=[
            pltpu.VMEM_SHARED((ACC_ROWS, d), jnp.float32),
            pltpu.VMEM((nch, 1, CHUNK), jnp.int32),
            pltpu.VMEM((1, CHUNK), jnp.int32),
            pltpu.VMEM((1, CHUNK), jnp.int32),
            pltpu.VMEM((CHUNK, d), jnp.float32),
            pltpu.VMEM((CHUNK, d), jnp.float32),
            pltpu.SemaphoreType.DMA,
            pltpu.SemaphoreType.DMA,
            pltpu.SemaphoreType.DMA,
            pltpu.SemaphoreType.DMA,
        ],
    )
    def matvec_kernel(p_hbm, src_hbm, dst_hbm, zerosd_hbm, u_hbm,
                      acc, sbuf, dbi0, dbi1, rows0, rows1,
                      sem0, sem1, semd0, semd1):
        ci, sid, w = _tile_id()
        pltpu.sync_copy(zerosd_hbm.at[pl.ds(sid * RPT, RPT)],
                        acc.at[pl.ds(sid * RPT, RPT)])
        pltpu.sync_copy(src_hbm.at[pl.ds(w * nch, nch)], sbuf)
        plsc.subcore_barrier()

        def fire(c, dbi, rows, semg, semd):
            pltpu.async_copy(dst_hbm.at[w * nch + c], dbi, semd)
            pltpu.async_copy(p_hbm.at[sbuf.at[c, 0]], rows, semg)

        def wait_and_scatter(dbi, rows, semg, semd):
            pltpu.make_async_copy(p_hbm.at[sbuf.at[0, 0]], rows, semg).wait()
            pltpu.make_async_copy(dst_hbm.at[w * nch], dbi, semd).wait()
            pltpu.sync_copy(rows, acc.at[dbi.at[0]], add=True)

        npair = nch // 2
        fire(0, dbi0, rows0, sem0, semd0)

        @pl.loop(0, npair - 1)
        def _(pr):
            cA = pr * 2
            fire(cA + 1, dbi1, rows1, sem1, semd1)
            wait_and_scatter(dbi0, rows0, sem0, semd0)
            fire(cA + 2, dbi0, rows0, sem0, semd0)
            wait_and_scatter(dbi1, rows1, sem1, semd1)

        fire(nch - 1, dbi1, rows1, sem1, semd1)
        wait_and_scatter(dbi0, rows0, sem0, semd0)
        wait_and_scatter(dbi1, rows1, sem1, semd1)

        plsc.subcore_barrier()
        pltpu.sync_copy(acc.at[pl.ds(sid * RPT, RPT)],
                        u_hbm.at[ci, pl.ds(sid * RPT, RPT)])

    return matvec_kernel


def _make_deg_kernel(nch):
    @functools.partial(
        pl.kernel,
        out_type=jax.ShapeDtypeStruct((NC, ACC_ROWS, D_IN), jnp.float32),
        mesh=_MESH,
        scratch_types=[
            pltpu.VMEM_SHARED((ACC_ROWS, D_IN), jnp.float32),
            pltpu.VMEM((CHUNK, D_IN), jnp.float32),
            pltpu.VMEM((1, CHUNK), jnp.int32),
            pltpu.VMEM((1, CHUNK), jnp.int32),
            pltpu.SemaphoreType.DMA,
            pltpu.SemaphoreType.DMA,
        ],
    )
    def deg_kernel(idx_hbm, ones_hbm, zerosd_hbm, u_hbm,
                   acc, ones_v, dbi0, dbi1, semd0, semd1):
        ci, sid, w = _tile_id()
        pltpu.sync_copy(zerosd_hbm.at[pl.ds(sid * RPT, RPT)],
                        acc.at[pl.ds(sid * RPT, RPT)])
        pltpu.sync_copy(ones_hbm, ones_v)
        plsc.subcore_barrier()

        def fire(c, dbi, semd):
            pltpu.async_copy(idx_hbm.at[w * nch + c], dbi, semd)

        def wait_and_scatter(dbi, semd):
            pltpu.make_async_copy(idx_hbm.at[w * nch], dbi, semd).wait()
            pltpu.sync_copy(ones_v, acc.at[dbi.at[0]], add=True)

        npair = nch // 2
        fire(0, dbi0, semd0)

        @pl.loop(0, npair - 1)
        def _(pr):
            cA = pr * 2
            fire(cA + 1, dbi1, semd1)
            wait_and_scatter(dbi0, semd0)
            fire(cA + 2, dbi0, semd0)
            wait_and_scatter(dbi1, semd1)

        fire(nch - 1, dbi1, semd1)
        wait_and_scatter(dbi0, semd0)
        wait_and_scatter(dbi1, semd1)

        plsc.subcore_barrier()
        pltpu.sync_copy(acc.at[pl.ds(sid * RPT, RPT)],
                        u_hbm.at[ci, pl.ds(sid * RPT, RPT)])

    return deg_kernel


_RB = 1000
_GRID = (N // _RB,)


def _step0(x, degtab, w0):
    def body(x_ref, d0_ref, d1_ref, w_ref, g_ref, p_ref, o_ref):
        deg = d0_ref[0, :, 0:1] + d1_ref[0, :, 0:1]
        g = jnp.where(deg > 0.0, lax.rsqrt(jnp.maximum(deg, 1e-30)), 0.0)
        xb = x_ref[...]
        g_ref[...] = jnp.broadcast_to(g, (_RB, D_IN))
        p_ref[...] = g * xb
        o_ref[...] = jnp.dot(xb, w_ref[...], preferred_element_type=jnp.float32)

    return pl.pallas_call(
        body,
        grid=_GRID,
        in_specs=[
            pl.BlockSpec((_RB, D_IN), lambda i: (i, 0)),
            pl.BlockSpec((1, _RB, D_IN), lambda i: (0, i, 0)),
            pl.BlockSpec((1, _RB, D_IN), lambda i: (1, i, 0)),
            pl.BlockSpec((D_IN, D_H), lambda i: (0, 0)),
        ],
        out_specs=[
            pl.BlockSpec((_RB, D_IN), lambda i: (i, 0)),
            pl.BlockSpec((_RB, D_IN), lambda i: (i, 0)),
            pl.BlockSpec((_RB, D_H), lambda i: (i, 0)),
        ],
        out_shape=[
            jax.ShapeDtypeStruct((N, D_IN), jnp.float32),
            jax.ShapeDtypeStruct((N, D_IN), jnp.float32),
            jax.ShapeDtypeStruct((N, D_H), jnp.float32),
        ],
    )(x, degtab, degtab, w0)


def _step_k(u, g, tprev, wk, out_in, c, has_prev, is_last, bias=None):
    def body(*refs):
        if is_last:
            u0_ref, u1_ref, g_ref, tp_ref, w_ref, b_ref, oi_ref, o_ref = refs
        else:
            u0_ref, u1_ref, g_ref, tp_ref, w_ref, oi_ref, t_ref, p_ref, o_ref = refs
        ub = u0_ref[0] + u1_ref[0]
        t = (-float(c)) * g_ref[...] * ub
        if has_prev:
            t = t - tp_ref[...]
        acc = oi_ref[...] + jnp.dot(t, refs[4][...],
                                    preferred_element_type=jnp.float32)
        if is_last:
            o_ref[...] = jnp.maximum(acc + b_ref[...], 0.0)
        else:
            t_ref[...] = t
            p_ref[...] = g_ref[...] * t
            o_ref[...] = acc

    in_specs = [
        pl.BlockSpec((1, _RB, D_IN), lambda i: (0, i, 0)),
        pl.BlockSpec((1, _RB, D_IN), lambda i: (1, i, 0)),
        pl.BlockSpec((_RB, D_IN), lambda i: (i, 0)),
        pl.BlockSpec((_RB, D_IN), lambda i: (i, 0)),
        pl.BlockSpec((D_IN, D_H), lambda i: (0, 0)),
    ]
    args = [u, u, g, tprev, wk]
    if is_last:
        in_specs.append(pl.BlockSpec((1, D_H), lambda i: (0, 0)))
        args.append(bias)
    in_specs.append(pl.BlockSpec((_RB, D_H), lambda i: (i, 0)))
    args.append(out_in)

    out_specs = []
    out_shape = []
    if not is_last:
        out_specs += [pl.BlockSpec((_RB, D_IN), lambda i: (i, 0)),
                      pl.BlockSpec((_RB, D_IN), lambda i: (i, 0))]
        out_shape += [jax.ShapeDtypeStruct((N, D_IN), jnp.float32),
                      jax.ShapeDtypeStruct((N, D_IN), jnp.float32)]
    out_specs.append(pl.BlockSpec((_RB, D_H), lambda i: (i, 0)))
    out_shape.append(jax.ShapeDtypeStruct((N, D_H), jnp.float32))

    n_in = len(args)
    alias = {n_in - 1: len(out_shape) - 1}

    return pl.pallas_call(
        body,
        grid=_GRID,
        in_specs=in_specs,
        out_specs=out_specs,
        out_shape=out_shape,
        input_output_aliases=alias,
    )(*args)


def kernel(x, edge_index, W, b):
    E = edge_index.shape[1]
    per_tile = NW * CHUNK
    nch = -(-E // per_tile)
    nch = -(-nch // 8) * 8
    e_pad = nch * per_tile

    src = edge_index[0].astype(jnp.int32)
    dst = edge_index[1].astype(jnp.int32)
    pad = e_pad - E
    if pad:
        z = (jnp.arange(pad, dtype=jnp.int32) * 41) % N
        src = jnp.concatenate([src, z])
        dst = jnp.concatenate([dst, z])
    src2d = src.reshape(NW * nch, CHUNK)
    dst2d = dst.reshape(NW * nch, CHUNK)

    zerosd = jnp.zeros((ACC_ROWS, D_IN), jnp.float32)
    ones_blk = jnp.ones((CHUNK, D_IN), jnp.float32)

    seff2d, deff2d = _edge_prep(src2d, dst2d)
    seff3d = seff2d.reshape(NW * nch, 1, CHUNK)
    deff3d = deff2d.reshape(NW * nch, 1, CHUNK)
    src3d = src2d.reshape(NW * nch, 1, CHUNK)

    matvec = _make_matvec_kernel(nch, D_IN)
    degu = _make_deg_kernel(nch)(seff3d, ones_blk, zerosd)

    g, p, out = _step0(x, degu, W[0])
    tprev = x
    tcur = None
    for k in range(1, K):
        u = matvec(p, src3d, deff3d, zerosd)
        is_last = k == K - 1
        c = 1 if k == 1 else 2
        if is_last:
            (out,) = _step_k(u, g, tprev, W[k], out, c=c, has_prev=True,
                             is_last=True, bias=b.reshape(1, D_H))
        elif k == 1:
            tcur, p, out = _step_k(u, g, x, W[k], out, c=1, has_prev=False,
                                   is_last=False)
            tprev = x
        else:
            tnew, p, out = _step_k(u, g, tprev, W[k], out, c=2, has_prev=True,
                                   is_last=False)
            tprev = tcur
            tcur = tnew
    return out
